# Initial kernel scaffold; baseline (speedup 1.0000x reference)
import jax
import jax.numpy as jnp
from jax.experimental import pallas as pl


def kernel(x, emb):
    raise NotImplementedError("write your pallas kernel here")


# speedup vs baseline: 1.8431x; 1.8431x over previous
import functools

import jax
import jax.numpy as jnp
from jax import lax
from jax.experimental import pallas as pl
from jax.experimental.pallas import tpu as pltpu
from jax.experimental.pallas import tpu_sc as plsc

EMBED_DIM = 64
NUM_CORES = 2
NUM_SUBCORES = 16
NUM_WORKERS = NUM_CORES * NUM_SUBCORES


def _pick_chunk(b_per_w: int) -> int:
    for c in (1024, 800, 640, 512, 400, 320, 256, 200, 160, 128, 64, 32, 16, 8):
        if b_per_w % c == 0:
            return c
    return b_per_w


@functools.lru_cache(maxsize=None)
def _build(B: int, D: int):
    b_per_w = B // NUM_WORKERS
    C = _pick_chunk(b_per_w)
    n_chunks = b_per_w // C
    mesh = plsc.VectorSubcoreMesh(core_axis_name="c", subcore_axis_name="s")

    @functools.partial(
        pl.kernel,
        out_type=jax.ShapeDtypeStruct((B, D), jnp.float32),
        mesh=mesh,
        scratch_types=[
            pltpu.VMEM((C,), jnp.int32),
            pltpu.VMEM((C, D), jnp.float32),
            pltpu.SemaphoreType.DMA,
        ],
        compiler_params=pltpu.CompilerParams(use_tc_tiling_on_sc=False),
    )
    def gather_kernel(x_hbm, emb_hbm, out_hbm, idx_v, rows_v, sem):
        wid = lax.axis_index("s") * NUM_CORES + lax.axis_index("c")
        base = wid * b_per_w

        def step(g, carry):
            off = pl.multiple_of(base + g * C, 8)
            pltpu.sync_copy(x_hbm.at[pl.ds(off, C)], idx_v)
            pltpu.async_copy(emb_hbm.at[idx_v], rows_v, sem).wait()
            pltpu.sync_copy(rows_v, out_hbm.at[pl.ds(off, C)])
            return carry

        lax.fori_loop(0, n_chunks, step, 0)

    return gather_kernel


def kernel(x, emb):
    S, T = x.shape
    B = S * T
    D = emb.shape[1]
    xf = x.reshape(B).astype(jnp.int32)
    out = _build(B, D)(xf, emb)
    return out.reshape(S, T, D)


# speedup vs baseline: 1.8722x; 1.0158x over previous
import functools

import jax
import jax.numpy as jnp
from jax import lax
from jax.experimental import pallas as pl
from jax.experimental.pallas import tpu as pltpu
from jax.experimental.pallas import tpu_sc as plsc

EMBED_DIM = 64
NUM_CORES = 2
NUM_SUBCORES = 16
NUM_WORKERS = NUM_CORES * NUM_SUBCORES


def _pick_chunk(b_per_w: int) -> int:
    for c in (800, 640, 512, 400, 320, 256, 200, 160, 128, 64, 32, 16, 8):
        if b_per_w % c == 0 and (b_per_w // c) % 2 == 0:
            return c
    return b_per_w


@functools.lru_cache(maxsize=None)
def _build(B: int, D: int):
    b_per_w = B // NUM_WORKERS
    C = _pick_chunk(b_per_w)
    n_chunks = b_per_w // C
    mesh = plsc.VectorSubcoreMesh(core_axis_name="c", subcore_axis_name="s")

    @functools.partial(
        pl.kernel,
        out_type=jax.ShapeDtypeStruct((B, D), jnp.float32),
        mesh=mesh,
        scratch_types=[
            pltpu.VMEM((n_chunks, C), jnp.int32),
            pltpu.VMEM((C, D), jnp.float32),
            pltpu.VMEM((C, D), jnp.float32),
            pltpu.SemaphoreType.DMA,
            pltpu.SemaphoreType.DMA,
        ],
        compiler_params=pltpu.CompilerParams(use_tc_tiling_on_sc=False),
    )
    def gather_kernel(x_hbm, emb_hbm, out_hbm, idx_all, rows0, rows1, sem0, sem1):
        wid = lax.axis_index("s") * NUM_CORES + lax.axis_index("c")
        base = wid * b_per_w
        rows = (rows0, rows1)
        sems = (sem0, sem1)

        pltpu.sync_copy(x_hbm.at[wid], idx_all)

        pltpu.async_copy(emb_hbm.at[idx_all.at[0]], rows0, sem0)

        def step(i, carry):
            g0 = i * 2
            for b in range(2):
                g = g0 + b
                nxt = g + 1

                @pl.when(nxt < n_chunks)
                def _():
                    pltpu.async_copy(
                        emb_hbm.at[idx_all.at[nxt]], rows[1 - b], sems[1 - b]
                    )

                pltpu.make_async_copy(
                    emb_hbm.at[idx_all.at[g]], rows[b], sems[b]
                ).wait()
                pltpu.sync_copy(rows[b], out_hbm.at[pl.ds(base + g * C, C)])
            return carry

        lax.fori_loop(0, n_chunks // 2, step, 0)

    return gather_kernel


def kernel(x, emb):
    S, T = x.shape
    B = S * T
    D = emb.shape[1]
    b_per_w = B // NUM_WORKERS
    C = _pick_chunk(b_per_w)
    x3 = x.reshape(NUM_WORKERS, b_per_w // C, C).astype(jnp.int32)
    out = _build(B, D)(x3, emb)
    return out.reshape(S, T, D)
